# baseline (speedup 1.0000x reference)
import jax
import jax.numpy as jnp
from jax.experimental import pallas as pl
from jax.experimental.pallas import tpu as pltpu

N = 65536
H = 1024
D = 512
STATE_DIM = 3
NT = 4096


def _dot(a, b, out_dtype=jnp.float32):
    return jax.lax.dot_general(
        a, b,
        dimension_numbers=(((1,), (0,)), ((), ())),
        preferred_element_type=out_dtype,
    )


def _mlp_kernel(pcA_ref, W1A_ref, W2T_ref, b2_ref, WlT_ref, bl_ref,
                mu_ref, lam_ref):
    hT = jnp.maximum(_dot(W1A_ref[...], pcA_ref[...]),
                     0.0).astype(jnp.bfloat16)
    mu_ref[...] = _dot(W2T_ref[...], hT) + b2_ref[...]
    l = jnp.tanh(_dot(WlT_ref[...], hT) + bl_ref[...])
    lam_ref[...] = jnp.concatenate(
        [l, jnp.zeros((1, l.shape[1]), jnp.float32)], axis=0)


def kernel(point_cloud, W1, b1, W2, b2, Wl, bl):
    pcA = jnp.concatenate(
        [point_cloud.T, jnp.ones((1, N), jnp.float32)], axis=0)
    W1A = jnp.concatenate([W1, b1[None, :]], axis=0).T
    W2T = W2.T.astype(jnp.bfloat16)
    WlT = Wl.T.astype(jnp.bfloat16)
    b2c = b2[:, None]
    blc = bl[:, None]

    grid = (N // NT,)
    mu, lam = pl.pallas_call(
        _mlp_kernel,
        grid=grid,
        in_specs=[
            pl.BlockSpec((3, NT), lambda i: (0, i)),
            pl.BlockSpec((H, 3), lambda i: (0, 0)),
            pl.BlockSpec((D, H), lambda i: (0, 0)),
            pl.BlockSpec((D, 1), lambda i: (0, 0)),
            pl.BlockSpec((2, H), lambda i: (0, 0)),
            pl.BlockSpec((2, 1), lambda i: (0, 0)),
        ],
        out_specs=[
            pl.BlockSpec((D, NT), lambda i: (0, i)),
            pl.BlockSpec((STATE_DIM, NT), lambda i: (0, i)),
        ],
        out_shape=[
            jax.ShapeDtypeStruct((D, N), jnp.float32),
            jax.ShapeDtypeStruct((STATE_DIM, N), jnp.float32),
        ],
        compiler_params=pltpu.CompilerParams(
            dimension_semantics=("parallel",)),
    )(pcA, W1A, W2T, b2c, WlT, blc)
    return (mu, lam)


# speedup vs baseline: 1.2551x; 1.2551x over previous
import jax
import jax.numpy as jnp
from jax.experimental import pallas as pl
from jax.experimental.pallas import tpu as pltpu

N = 65536
H = 1024
D = 512
STATE_DIM = 3
NT = 4096


def _dot(a, b):
    return jax.lax.dot_general(
        a, b,
        dimension_numbers=(((1,), (0,)), ((), ())),
        preferred_element_type=jnp.float32,
    )


def _mlp_kernel(ptsT_ref, W1T_ref, b1_ref, W2T_ref, b2_ref, WlT_ref, bl_ref,
                mu_ref, lam_ref):
    x0 = ptsT_ref[0:1, :]
    x1 = ptsT_ref[1:2, :]
    w10 = W1T_ref[:, 0:1]
    w11 = W1T_ref[:, 1:2]
    hT = jnp.maximum(w10 * x0 + w11 * x1 + b1_ref[...],
                     0.0).astype(jnp.bfloat16)
    mu_ref[...] = _dot(W2T_ref[...], hT) + b2_ref[...]
    l = jnp.tanh(_dot(WlT_ref[...], hT) + bl_ref[...])
    lam_ref[...] = jnp.concatenate(
        [l, jnp.zeros((1, l.shape[1]), jnp.float32)], axis=0)


def kernel(point_cloud, W1, b1, W2, b2, Wl, bl):
    ptsT = point_cloud.T
    W1T = W1.T
    W2T = W2.T.astype(jnp.bfloat16)
    WlT = Wl.T.astype(jnp.bfloat16)
    b1c = b1[:, None]
    b2c = b2[:, None]
    blc = bl[:, None]

    grid = (N // NT,)
    mu, lam = pl.pallas_call(
        _mlp_kernel,
        grid=grid,
        in_specs=[
            pl.BlockSpec((2, NT), lambda i: (0, i)),
            pl.BlockSpec((H, 2), lambda i: (0, 0)),
            pl.BlockSpec((H, 1), lambda i: (0, 0)),
            pl.BlockSpec((D, H), lambda i: (0, 0)),
            pl.BlockSpec((D, 1), lambda i: (0, 0)),
            pl.BlockSpec((2, H), lambda i: (0, 0)),
            pl.BlockSpec((2, 1), lambda i: (0, 0)),
        ],
        out_specs=[
            pl.BlockSpec((D, NT), lambda i: (0, i)),
            pl.BlockSpec((STATE_DIM, NT), lambda i: (0, i)),
        ],
        out_shape=[
            jax.ShapeDtypeStruct((D, N), jnp.float32),
            jax.ShapeDtypeStruct((STATE_DIM, N), jnp.float32),
        ],
        compiler_params=pltpu.CompilerParams(
            dimension_semantics=("parallel",)),
    )(ptsT, W1T, b1c, W2T, b2c, WlT, blc)
    return (mu, lam)


# speedup vs baseline: 1.2696x; 1.0115x over previous
import jax
import jax.numpy as jnp
from jax.experimental import pallas as pl
from jax.experimental.pallas import tpu as pltpu

N = 65536
H = 1024
D = 512
STATE_DIM = 3
NT = 4096


def _dot(a, b):
    return jax.lax.dot_general(
        a, b,
        dimension_numbers=(((1,), (0,)), ((), ())),
        preferred_element_type=jnp.float32,
    )


def _mlp_kernel(ptsT_ref, W1T_ref, b1_ref, W2T_ref, b2_ref, WlT_ref, bl_ref,
                mu_ref, lam_ref):
    x0 = ptsT_ref[0:1, :].astype(jnp.bfloat16)
    x1 = ptsT_ref[1:2, :].astype(jnp.bfloat16)
    w10 = W1T_ref[:, 0:1].astype(jnp.bfloat16)
    w11 = W1T_ref[:, 1:2].astype(jnp.bfloat16)
    b1 = b1_ref[...].astype(jnp.bfloat16)
    hT = jnp.maximum(w10 * x0 + w11 * x1 + b1,
                     jnp.bfloat16(0.0))
    mu_ref[...] = _dot(W2T_ref[...], hT) + b2_ref[...]
    l = jnp.tanh(_dot(WlT_ref[...], hT) + bl_ref[...])
    lam_ref[...] = jnp.concatenate(
        [l, jnp.zeros((1, l.shape[1]), jnp.float32)], axis=0)


def kernel(point_cloud, W1, b1, W2, b2, Wl, bl):
    ptsT = point_cloud.T
    W1T = W1.T
    W2T = W2.T.astype(jnp.bfloat16)
    WlT = Wl.T.astype(jnp.bfloat16)
    b1c = b1[:, None]
    b2c = b2[:, None]
    blc = bl[:, None]

    grid = (N // NT,)
    mu, lam = pl.pallas_call(
        _mlp_kernel,
        grid=grid,
        in_specs=[
            pl.BlockSpec((2, NT), lambda i: (0, i)),
            pl.BlockSpec((H, 2), lambda i: (0, 0)),
            pl.BlockSpec((H, 1), lambda i: (0, 0)),
            pl.BlockSpec((D, H), lambda i: (0, 0)),
            pl.BlockSpec((D, 1), lambda i: (0, 0)),
            pl.BlockSpec((2, H), lambda i: (0, 0)),
            pl.BlockSpec((2, 1), lambda i: (0, 0)),
        ],
        out_specs=[
            pl.BlockSpec((D, NT), lambda i: (0, i)),
            pl.BlockSpec((STATE_DIM, NT), lambda i: (0, i)),
        ],
        out_shape=[
            jax.ShapeDtypeStruct((D, N), jnp.float32),
            jax.ShapeDtypeStruct((STATE_DIM, N), jnp.float32),
        ],
        compiler_params=pltpu.CompilerParams(
            dimension_semantics=("parallel",)),
    )(ptsT, W1T, b1c, W2T, b2c, WlT, blc)
    return (mu, lam)


# speedup vs baseline: 1.2826x; 1.0103x over previous
import jax
import jax.numpy as jnp
from jax.experimental import pallas as pl
from jax.experimental.pallas import tpu as pltpu

N = 65536
H = 1024
D = 512
STATE_DIM = 3
NT = 8192


def _dot(a, b):
    return jax.lax.dot_general(
        a, b,
        dimension_numbers=(((1,), (0,)), ((), ())),
        preferred_element_type=jnp.float32,
    )


def _mlp_kernel(ptsT_ref, W1T_ref, b1_ref, W2T_ref, b2_ref, WlT_ref, bl_ref,
                mu_ref, lam_ref):
    x0 = ptsT_ref[0:1, :].astype(jnp.bfloat16)
    x1 = ptsT_ref[1:2, :].astype(jnp.bfloat16)
    w10 = W1T_ref[:, 0:1].astype(jnp.bfloat16)
    w11 = W1T_ref[:, 1:2].astype(jnp.bfloat16)
    b1 = b1_ref[...].astype(jnp.bfloat16)
    hT = jnp.maximum(w10 * x0 + w11 * x1 + b1,
                     jnp.bfloat16(0.0))
    mu_ref[...] = _dot(W2T_ref[...], hT) + b2_ref[...]
    l = jnp.tanh(_dot(WlT_ref[...], hT) + bl_ref[...])
    lam_ref[...] = jnp.concatenate(
        [l, jnp.zeros((1, l.shape[1]), jnp.float32)], axis=0)


def kernel(point_cloud, W1, b1, W2, b2, Wl, bl):
    ptsT = point_cloud.T
    W1T = W1.T
    W2T = W2.T.astype(jnp.bfloat16)
    WlT = Wl.T.astype(jnp.bfloat16)
    b1c = b1[:, None]
    b2c = b2[:, None]
    blc = bl[:, None]

    grid = (N // NT,)
    mu, lam = pl.pallas_call(
        _mlp_kernel,
        grid=grid,
        in_specs=[
            pl.BlockSpec((2, NT), lambda i: (0, i)),
            pl.BlockSpec((H, 2), lambda i: (0, 0)),
            pl.BlockSpec((H, 1), lambda i: (0, 0)),
            pl.BlockSpec((D, H), lambda i: (0, 0)),
            pl.BlockSpec((D, 1), lambda i: (0, 0)),
            pl.BlockSpec((2, H), lambda i: (0, 0)),
            pl.BlockSpec((2, 1), lambda i: (0, 0)),
        ],
        out_specs=[
            pl.BlockSpec((D, NT), lambda i: (0, i)),
            pl.BlockSpec((STATE_DIM, NT), lambda i: (0, i)),
        ],
        out_shape=[
            jax.ShapeDtypeStruct((D, N), jnp.float32),
            jax.ShapeDtypeStruct((STATE_DIM, N), jnp.float32),
        ],
        compiler_params=pltpu.CompilerParams(
            dimension_semantics=("parallel",)),
    )(ptsT, W1T, b1c, W2T, b2c, WlT, blc)
    return (mu, lam)
